# Initial kernel scaffold; baseline (speedup 1.0000x reference)
import jax
import jax.numpy as jnp
from jax.experimental import pallas as pl


def kernel(x, edge_index, W1, att_src1, att_dst1, b1, W2, att_src2, att_dst2, b2):
    raise NotImplementedError("write your pallas kernel here")


# speedup vs baseline: 40.6673x; 40.6673x over previous
import functools

import jax
import jax.numpy as jnp
from jax import lax
from jax.experimental import pallas as pl
from jax.experimental.pallas import tpu as pltpu
from jax.experimental.pallas import tpu_sc as plsc

N = 10000
E = 320000
NPAD = 10112
CH = 128
NTILES = 32
EPAD = 331776
CPT = EPAD // (NTILES * CH)
EPT = CPT * CH
RPT = NPAD // 16
EPS = 1e-16

_MESH = plsc.VectorSubcoreMesh(core_axis_name="c", subcore_axis_name="s")


def _lane_iota():
    return lax.broadcasted_iota(jnp.int32, (16,), 0)


def _sc_gat1(h_hbm, as_hbm, ad_hbm, src_hbm, dst_hbm, zacc_hbm, zden_hbm,
             acc_out, den_out,
             sidx, didx, asb, adb, hb, eab, eabf, accS, denS, s0, s1, s2):
    c = lax.axis_index("c")
    s = lax.axis_index("s")
    wid = c * 16 + s

    r0 = s * RPT
    pltpu.sync_copy(zacc_hbm.at[pl.ds(r0, RPT)], accS.at[pl.ds(r0, RPT)])
    pltpu.sync_copy(zden_hbm.at[pl.ds(r0, RPT)], denS.at[pl.ds(r0, RPT)])
    plsc.subcore_barrier()

    e0 = wid * EPT

    @pl.loop(0, CPT)
    def _chunk(t):
        base = e0 + t * CH
        pltpu.sync_copy(src_hbm.at[pl.ds(base, CH)], sidx)
        pltpu.sync_copy(dst_hbm.at[pl.ds(base, CH)], didx)
        ga = pltpu.async_copy(as_hbm.at[sidx], asb, s0)
        gb = pltpu.async_copy(ad_hbm.at[didx], adb, s1)
        gh = pltpu.async_copy(h_hbm.at[sidx], hb, s2)
        ga.wait()
        gb.wait()
        gh.wait()

        @pl.loop(0, CH)
        def _edge(e):
            al = asb[e, :] + adb[e, :]
            ea = jnp.exp(jnp.maximum(al, 0.2 * al))
            eab[e, :] = ea
            eabf[pl.ds(e * 16, 16)] = ea
            for k in range(8):
                w = plsc.load_gather(eabf, [jnp.full((16,), e * 16 + k, jnp.int32)])
                hv = hb[e, pl.ds(k * 16, 16)]
                hb[e, pl.ds(k * 16, 16)] = hv * w

        pltpu.sync_copy(hb, accS.at[didx], add=True)
        pltpu.sync_copy(eab, denS.at[didx], add=True)

    plsc.subcore_barrier()
    pltpu.sync_copy(accS.at[pl.ds(r0, RPT)], acc_out.at[c, pl.ds(r0, RPT)])
    pltpu.sync_copy(denS.at[pl.ds(r0, RPT)], den_out.at[c, pl.ds(r0, RPT)])


_sc_gat1_call = functools.partial(
    pl.kernel,
    out_type=[
        jax.ShapeDtypeStruct((2, NPAD, 128), jnp.float32),
        jax.ShapeDtypeStruct((2, NPAD, 16), jnp.float32),
    ],
    mesh=_MESH,
    scratch_types=[
        pltpu.VMEM((CH,), jnp.int32),
        pltpu.VMEM((CH,), jnp.int32),
        pltpu.VMEM((CH, 16), jnp.float32),
        pltpu.VMEM((CH, 16), jnp.float32),
        pltpu.VMEM((CH, 128), jnp.float32),
        pltpu.VMEM((CH, 16), jnp.float32),
        pltpu.VMEM((CH * 16,), jnp.float32),
        pltpu.VMEM_SHARED((NPAD, 128), jnp.float32),
        pltpu.VMEM_SHARED((NPAD, 16), jnp.float32),
        pltpu.SemaphoreType.DMA,
        pltpu.SemaphoreType.DMA,
        pltpu.SemaphoreType.DMA,
    ],
    compiler_params=pltpu.CompilerParams(needs_layout_passes=False, use_tc_tiling_on_sc=False),
)(_sc_gat1)


def _sc_gat2(y_hbm, a2_hbm, src_hbm, dst_hbm, zacc_hbm,
             acc_out,
             sidx, didx, a2t, yb, eab, accS, s0, s1):
    c = lax.axis_index("c")
    s = lax.axis_index("s")
    wid = c * 16 + s

    r0 = s * RPT
    pltpu.sync_copy(zacc_hbm.at[pl.ds(r0, RPT)], accS.at[pl.ds(r0, RPT)])
    pltpu.sync_copy(a2_hbm, a2t)
    plsc.subcore_barrier()

    e0 = wid * EPT

    @pl.loop(0, CPT)
    def _chunk(t):
        base = e0 + t * CH
        pltpu.sync_copy(src_hbm.at[pl.ds(base, CH)], sidx)
        pltpu.sync_copy(dst_hbm.at[pl.ds(base, CH)], didx)
        gy = pltpu.async_copy(y_hbm.at[sidx], yb, s0)

        @pl.loop(0, CH // 16)
        def _grp(g):
            sv = sidx[pl.ds(g * 16, 16)]
            dv = didx[pl.ds(g * 16, 16)]
            vs = plsc.load_gather(a2t, [sv * 2])
            vd = plsc.load_gather(a2t, [dv * 2 + 1])
            al = vs + vd
            eab[pl.ds(g * 16, 16)] = jnp.exp(jnp.maximum(al, 0.2 * al))

        gy.wait()

        @pl.loop(0, CH)
        def _edge(e):
            w = plsc.load_gather(eab, [jnp.full((16,), e, jnp.int32)])
            for j in range(3):
                yv = yb[e, pl.ds(j * 16, 16)]
                yb[e, pl.ds(j * 16, 16)] = yv * w

        pltpu.sync_copy(yb, accS.at[didx], add=True)

    plsc.subcore_barrier()
    pltpu.sync_copy(accS.at[pl.ds(r0, RPT)], acc_out.at[c, pl.ds(r0, RPT)])


_sc_gat2_call = functools.partial(
    pl.kernel,
    out_type=[jax.ShapeDtypeStruct((2, NPAD, 48), jnp.float32)],
    mesh=_MESH,
    scratch_types=[
        pltpu.VMEM((CH,), jnp.int32),
        pltpu.VMEM((CH,), jnp.int32),
        pltpu.VMEM((NPAD * 2,), jnp.float32),
        pltpu.VMEM((CH, 48), jnp.float32),
        pltpu.VMEM((CH,), jnp.float32),
        pltpu.VMEM_SHARED((NPAD, 48), jnp.float32),
        pltpu.SemaphoreType.DMA,
        pltpu.SemaphoreType.DMA,
    ],
    compiler_params=pltpu.CompilerParams(needs_layout_passes=False, use_tc_tiling_on_sc=False),
)(_sc_gat2)


BR = 1264


def _tc1_body(x_ref, w_ref, ps_ref, pd_ref, h_ref, as_ref, ad_ref):
    h = jnp.dot(x_ref[...], w_ref[...], preferred_element_type=jnp.float32)
    h_ref[...] = h
    as_ref[...] = jnp.dot(h, ps_ref[...], preferred_element_type=jnp.float32)
    ad_ref[...] = jnp.dot(h, pd_ref[...], preferred_element_type=jnp.float32)


def _tc1(xp, W1, Ps, Pd):
    return pl.pallas_call(
        _tc1_body,
        grid=(NPAD // BR,),
        in_specs=[
            pl.BlockSpec((BR, 128), lambda i: (i, 0)),
            pl.BlockSpec((128, 128), lambda i: (0, 0)),
            pl.BlockSpec((128, 16), lambda i: (0, 0)),
            pl.BlockSpec((128, 16), lambda i: (0, 0)),
        ],
        out_specs=[
            pl.BlockSpec((BR, 128), lambda i: (i, 0)),
            pl.BlockSpec((BR, 16), lambda i: (i, 0)),
            pl.BlockSpec((BR, 16), lambda i: (i, 0)),
        ],
        out_shape=[
            jax.ShapeDtypeStruct((NPAD, 128), jnp.float32),
            jax.ShapeDtypeStruct((NPAD, 16), jnp.float32),
            jax.ShapeDtypeStruct((NPAD, 16), jnp.float32),
        ],
    )(xp, W1, Ps, Pd)


def _tc2_body(acc_ref, den_ref, rm_ref, w2_ref, wsd_ref, b1_ref, y_ref, a2_ref):
    acc = acc_ref[0] + acc_ref[1]
    den8 = den_ref[0][:, :8] + den_ref[1][:, :8]
    denb = jnp.dot(den8, rm_ref[...], preferred_element_type=jnp.float32)
    h = jnp.maximum(acc / (denb + EPS) + b1_ref[...], 0.0)
    y = jnp.dot(h, w2_ref[...], preferred_element_type=jnp.float32)
    col = lax.broadcasted_iota(jnp.int32, (BR, 48), 1)
    y_ref[...] = y + jnp.where(col == 40, 1.0, 0.0)
    a2_ref[...] = jnp.dot(h, wsd_ref[...], preferred_element_type=jnp.float32)


def _tc2(acc1, den1, Rm, W2p, Wsd, b1):
    return pl.pallas_call(
        _tc2_body,
        grid=(NPAD // BR,),
        in_specs=[
            pl.BlockSpec((2, BR, 128), lambda i: (0, i, 0)),
            pl.BlockSpec((2, BR, 16), lambda i: (0, i, 0)),
            pl.BlockSpec((8, 128), lambda i: (0, 0)),
            pl.BlockSpec((128, 48), lambda i: (0, 0)),
            pl.BlockSpec((128, 2), lambda i: (0, 0)),
            pl.BlockSpec((1, 128), lambda i: (0, 0)),
        ],
        out_specs=[
            pl.BlockSpec((BR, 48), lambda i: (i, 0)),
            pl.BlockSpec((BR, 2), lambda i: (i, 0)),
        ],
        out_shape=[
            jax.ShapeDtypeStruct((NPAD, 48), jnp.float32),
            jax.ShapeDtypeStruct((NPAD, 2), jnp.float32),
        ],
    )(acc1, den1, Rm, W2p, Wsd, b1)


def _tc3_body(acc_ref, s_ref, b2_ref, out_ref):
    accs = acc_ref[0] + acc_ref[1]
    denb = jnp.dot(accs, s_ref[...], preferred_element_type=jnp.float32)
    out_ref[...] = accs / (denb + EPS) + b2_ref[...]


def _tc3(acc2, S, b2p):
    return pl.pallas_call(
        _tc3_body,
        grid=(NPAD // BR,),
        in_specs=[
            pl.BlockSpec((2, BR, 48), lambda i: (0, i, 0)),
            pl.BlockSpec((48, 48), lambda i: (0, 0)),
            pl.BlockSpec((1, 48), lambda i: (0, 0)),
        ],
        out_specs=pl.BlockSpec((BR, 48), lambda i: (i, 0)),
        out_shape=jax.ShapeDtypeStruct((NPAD, 48), jnp.float32),
    )(acc2, S, b2p)


def kernel(x, edge_index, W1, att_src1, att_dst1, b1, W2, att_src2, att_dst2, b2):
    f32 = jnp.float32
    i32 = jnp.int32

    loop = jnp.arange(N, dtype=i32)
    padlen = EPAD - E - N
    srcp = jnp.concatenate([edge_index[0].astype(i32), loop,
                            jnp.full((padlen,), N, i32)])
    dstp = jnp.concatenate([edge_index[1].astype(i32), loop,
                            jnp.full((padlen,), N, i32)])

    xp = jnp.zeros((NPAD, 128), f32).at[:N].set(x)

    Ps = jnp.zeros((128, 16), f32)
    Pd = jnp.zeros((128, 16), f32)
    for k in range(8):
        Ps = Ps.at[k * 16:(k + 1) * 16, k].set(att_src1[k])
        Ps = Ps.at[k * 16:(k + 1) * 16, 8 + k].set(att_src1[k])
        Pd = Pd.at[k * 16:(k + 1) * 16, k].set(att_dst1[k])
        Pd = Pd.at[k * 16:(k + 1) * 16, 8 + k].set(att_dst1[k])

    Rm = jnp.kron(jnp.eye(8, dtype=f32), jnp.ones((1, 16), f32))
    W2p = jnp.zeros((128, 48), f32).at[:, :40].set(W2)
    Wsd = jnp.stack([W2 @ att_src2[0], W2 @ att_dst2[0]], axis=1)
    S = jnp.zeros((48, 48), f32).at[40, :].set(1.0)
    b1p = b1.reshape(1, 128)
    b2p = jnp.zeros((1, 48), f32).at[0, :40].set(b2)

    zacc1 = jnp.zeros((NPAD, 128), f32)
    zden1 = jnp.zeros((NPAD, 16), f32)
    zacc2 = jnp.zeros((NPAD, 48), f32)

    h1, A1s, A1d = _tc1(xp, W1, Ps, Pd)
    acc1, den1 = _sc_gat1_call(h1, A1s, A1d, srcp, dstp, zacc1, zden1)
    Y, A2 = _tc2(acc1, den1, Rm, W2p, Wsd, b1p)
    acc2, = _sc_gat2_call(Y, A2.reshape(-1), srcp, dstp, zacc2)
    outp = _tc3(acc2, S, b2p)
    return outp[:N, :40]
